# baseline (speedup 1.0000x reference)
import jax
import jax.numpy as jnp
from jax import lax
from jax.experimental import pallas as pl
from jax.experimental.pallas import tpu as pltpu
from jax.experimental.pallas import tpu_sc as plsc

BATCH = 1024
MAXLEN = 200
N_FEAT = 64
CHUNK = 128
N_FLAT = BATCH * MAXLEN
N_CHUNKS = N_FLAT // CHUNK
NBUF = 10
DEPTH = 8


def _emb_body(x_hbm, pe_hbm, E_hbm, out_hbm, idx_v, rows_v, pe_v,
              sems_g, sems_w):
    info = plsc.get_sparse_core_info()
    nc, ns = info.num_cores, info.num_subcores
    nw = nc * ns
    wid = lax.axis_index("s") * nc + lax.axis_index("c")
    chunks_per_w = N_CHUNKS // nw
    cbase = wid * chunks_per_w

    pltpu.sync_copy(pe_hbm, pe_v)
    pltpu.sync_copy(x_hbm.at[pl.ds(cbase, chunks_per_w)], idx_v)

    def g_copies(c, u):
        cps = []
        for k in range(CHUNK // 16):
            idx16 = idx_v[c, pl.ds(k * 16, 16)]
            cps.append(pltpu.make_async_copy(
                E_hbm.at[idx16], rows_v.at[u].at[pl.ds(k * 16, 16)],
                sems_g[u]))
        return cps

    def w_copy(c, u):
        return pltpu.make_async_copy(
            rows_v.at[u], out_hbm.at[cbase + c], sems_w[u])

    def g_start(c, u):
        for cp in g_copies(c, u):
            cp.start()

    for d in range(DEPTH):
        g_start(d, d)

    def group(g, carry):
        for u in range(NBUF):
            c = g * NBUF + u
            nxt = (u + DEPTH) % NBUF

            @pl.when(c + DEPTH < chunks_per_w)
            def _():
                @pl.when(c >= NBUF - DEPTH)
                def _():
                    w_copy(c + DEPTH - NBUF, nxt).wait()
                g_start(c + DEPTH, nxt)

            for cp in g_copies(c, u):
                cp.wait()

            rows_b = rows_v.at[u]
            t0 = lax.rem(c * CHUNK, MAXLEN)

            @plsc.parallel_loop(0, CHUNK, step=1, unroll=4)
            def _(r):
                t = t0 + r
                t = jnp.where(t >= MAXLEN, t - MAXLEN, t)
                for cc in range(N_FEAT // 16):
                    sl = pl.ds(cc * 16, 16)
                    rows_b[r, sl] = rows_b[r, sl] + pe_v[t, sl]

            w_copy(c, u).start()
        return carry

    lax.fori_loop(0, chunks_per_w // NBUF, group, 0)

    for u in range(NBUF):
        c = chunks_per_w - NBUF + u
        w_copy(c, c % NBUF).wait()


def kernel(x, E, pe):
    pe2 = pe.reshape(MAXLEN, N_FEAT)
    x3 = x.reshape(N_CHUNKS, CHUNK)
    mesh = plsc.VectorSubcoreMesh(core_axis_name="c", subcore_axis_name="s")
    f = pl.kernel(
        _emb_body,
        out_type=jax.ShapeDtypeStruct((N_CHUNKS, CHUNK, N_FEAT), jnp.float32),
        mesh=mesh,
        compiler_params=pltpu.CompilerParams(use_tc_tiling_on_sc=False),
        scratch_types=[
            pltpu.VMEM((N_CHUNKS // 32, CHUNK), jnp.int32),
            pltpu.VMEM((NBUF, CHUNK, N_FEAT), jnp.float32),
            pltpu.VMEM((MAXLEN, N_FEAT), jnp.float32),
            [pltpu.SemaphoreType.DMA] * NBUF,
            [pltpu.SemaphoreType.DMA] * NBUF,
        ],
    )
    out = f(x3, pe2, E)
    return out.reshape(BATCH, MAXLEN, N_FEAT)
